# baseline (speedup 1.0000x reference)
import functools

import jax
import jax.numpy as jnp
from jax import lax
from jax.experimental import pallas as pl
from jax.experimental.pallas import tpu as pltpu
from jax.experimental.pallas import tpu_sc as plsc

B = 16384
E = 32
NC, NS = 2, 16
NW = NC * NS
BPW = B // NW
L = 16

RB = 2048
NBLK = B // RB


def _sc_gather_pair(ids_h, ta_t3, tb_t3):
  mesh = plsc.VectorSubcoreMesh(core_axis_name="c", subcore_axis_name="s")

  @functools.partial(
      pl.kernel,
      mesh=mesh,
      out_type=[jax.ShapeDtypeStruct((B, E), jnp.float32)] * 2,
      scratch_types=[
          pltpu.VMEM((BPW,), jnp.int32),
          pltpu.VMEM((BPW, E), jnp.float32),
          pltpu.SemaphoreType.DMA,
      ],
  )
  def k(ids_hb, ta_h, tb_h, oa_o, ob_o, ids_v, rows_v, sem):
    wid = lax.axis_index("s") * NC + lax.axis_index("c")
    base = wid * BPW
    pltpu.sync_copy(ids_hb.at[pl.ds(base, BPW)], ids_v)
    out_sl = pl.ds(base, BPW)
    for tab_h, out_h in ((ta_h, oa_o), (tb_h, ob_o)):
      def grp_body(g, _, tab_h=tab_h):
        idv = ids_v[pl.ds(g * L, L)]
        for l in range(L):
          idx = idv[l]
          t = lax.shift_right_logical(idx, 3)
          r = lax.bitwise_and(idx, 7)
          pltpu.async_copy(tab_h.at[t, r], rows_v.at[g * L + l], sem)
        return 0

      lax.fori_loop(0, BPW // L, grp_body, 0)
      pltpu.make_async_copy(out_h.at[out_sl], rows_v, sem).wait()
      pltpu.sync_copy(rows_v, out_h.at[out_sl])

  return k(ids_h, ta_t3, tb_t3)


def _sc_gather4(uid_h, iid_h, gu_t3, gi_t3, mu_t3, mi_t3):
  gu, mu = _sc_gather_pair(uid_h, gu_t3, mu_t3)
  gi, mi = _sc_gather_pair(iid_h, gi_t3, mi_t3)
  return gu, gi, mu, mi


def _tc_body(gu_r, gi_r, mu_r, mi_r, w1_r, b1_r, w2_r, b2_r, wf_r, bf_r, out_r):
  w1 = w1_r[...]
  h1 = lax.dot_general(mu_r[...], w1[:, :E], (((1,), (1,)), ((), ())),
                       preferred_element_type=jnp.float32)
  h1 = h1 + lax.dot_general(mi_r[...], w1[:, E:], (((1,), (1,)), ((), ())),
                            preferred_element_type=jnp.float32)
  h1 = jnp.maximum(h1 + b1_r[...], 0.0)
  h2 = lax.dot_general(h1, w2_r[...], (((1,), (1,)), ((), ())),
                       preferred_element_type=jnp.float32)
  h2 = jnp.maximum(h2 + b2_r[...], 0.0)
  gmf = gu_r[...] * gi_r[...]
  wf = wf_r[...]
  p = lax.dot_general(gmf, wf[:, :E], (((1,), (1,)), ((), ())),
                      preferred_element_type=jnp.float32)
  p = p + lax.dot_general(h2, wf[:, E:], (((1,), (1,)), ((), ())),
                          preferred_element_type=jnp.float32)
  out_r[...] = jax.nn.sigmoid(p + bf_r[...])


def _tc_mlp(gu, gi, mu, mi, W1, b1, W2, b2, Wf, bf):
  row = pl.BlockSpec((RB, E), lambda i: (i, 0))
  full = lambda a: pl.BlockSpec(a.shape, lambda i: (0,) * a.ndim)
  out = pl.pallas_call(
      _tc_body,
      grid=(NBLK,),
      in_specs=[row, row, row, row,
                full(W1), full(b1), full(W2), full(b2), full(Wf), full(bf)],
      out_specs=pl.BlockSpec((RB, 1), lambda i: (i, 0)),
      out_shape=jax.ShapeDtypeStruct((B, 1), jnp.float32),
  )(gu, gi, mu, mi, W1, b1, W2, b2, Wf, bf)
  return out.reshape(B)


def kernel(user_ids, item_ids, gmf_user_table, gmf_item_table,
           mlp_user_table, mlp_item_table, W1, b1, W2, b2, Wf, bf):
  uid = user_ids.astype(jnp.int32)
  iid = item_ids.astype(jnp.int32)
  t3 = lambda t: t.reshape(t.shape[0] // 8, 8, E)
  gu, gi, mu, mi = _sc_gather4(uid, iid,
                               t3(gmf_user_table), t3(gmf_item_table),
                               t3(mlp_user_table), t3(mlp_item_table))
  return _tc_mlp(gu, gi, mu, mi,
                 W1, b1.reshape(1, 64), W2, b2.reshape(1, 32),
                 Wf, bf.reshape(1, 1))


# speedup vs baseline: 1.0039x; 1.0039x over previous
import functools

import jax
import jax.numpy as jnp
from jax import lax
from jax.experimental import pallas as pl
from jax.experimental.pallas import tpu as pltpu
from jax.experimental.pallas import tpu_sc as plsc

B = 16384
E = 32
NC, NS = 2, 16
NW = NC * NS
BPW = B // NW
L = 16

RB = 2048
NBLK = B // RB


def _sc_gather4(uid_h, iid_h, gu_t3, gi_t3, mu_t3, mi_t3):
  mesh = plsc.VectorSubcoreMesh(core_axis_name="c", subcore_axis_name="s")

  @functools.partial(
      pl.kernel,
      mesh=mesh,
      out_type=[jax.ShapeDtypeStruct((B, E), jnp.float32)] * 4,
      scratch_types=[
          pltpu.VMEM((BPW,), jnp.int32),
          pltpu.VMEM((BPW,), jnp.int32),
          pltpu.VMEM((BPW, E), jnp.float32),
          pltpu.SemaphoreType.DMA,
      ],
  )
  def k(uid_hb, iid_hb, gut_h, git_h, mut_h, mit_h,
        gu_o, gi_o, mu_o, mi_o,
        uid_v, iid_v, rows_v, sem):
    wid = lax.axis_index("s") * NC + lax.axis_index("c")
    base = wid * BPW
    pltpu.sync_copy(uid_hb.at[pl.ds(base, BPW)], uid_v)
    pltpu.sync_copy(iid_hb.at[pl.ds(base, BPW)], iid_v)
    out_sl = pl.ds(base, BPW)
    for tab_h, ids_v, out_h in ((gut_h, uid_v, gu_o), (git_h, iid_v, gi_o),
                                (mut_h, uid_v, mu_o), (mit_h, iid_v, mi_o)):
      def grp_body(g, _, tab_h=tab_h, ids_v=ids_v):
        idv = ids_v[pl.ds(g * L, L)]
        for l in range(L):
          idx = idv[l]
          t = lax.shift_right_logical(idx, 3)
          r = lax.bitwise_and(idx, 7)
          pltpu.async_copy(tab_h.at[t, r], rows_v.at[g * L + l], sem)
        return 0

      lax.fori_loop(0, BPW // L, grp_body, 0)
      pltpu.make_async_copy(out_h.at[out_sl], rows_v, sem).wait()
      pltpu.sync_copy(rows_v, out_h.at[out_sl])

  return k(uid_h, iid_h, gu_t3, gi_t3, mu_t3, mi_t3)


def _tc_body(gu_r, gi_r, mu_r, mi_r, w1_r, b1_r, w2_r, b2_r, wf_r, bf_r, out_r):
  w1 = w1_r[...]
  h1 = lax.dot_general(mu_r[...], w1[:, :E], (((1,), (1,)), ((), ())),
                       preferred_element_type=jnp.float32)
  h1 = h1 + lax.dot_general(mi_r[...], w1[:, E:], (((1,), (1,)), ((), ())),
                            preferred_element_type=jnp.float32)
  h1 = jnp.maximum(h1 + b1_r[...], 0.0)
  h2 = lax.dot_general(h1, w2_r[...], (((1,), (1,)), ((), ())),
                       preferred_element_type=jnp.float32)
  h2 = jnp.maximum(h2 + b2_r[...], 0.0)
  gmf = gu_r[...] * gi_r[...]
  wf = wf_r[...]
  p = lax.dot_general(gmf, wf[:, :E], (((1,), (1,)), ((), ())),
                      preferred_element_type=jnp.float32)
  p = p + lax.dot_general(h2, wf[:, E:], (((1,), (1,)), ((), ())),
                          preferred_element_type=jnp.float32)
  out_r[...] = jax.nn.sigmoid(p + bf_r[...])


def _tc_mlp(gu, gi, mu, mi, W1, b1, W2, b2, Wf, bf):
  row = pl.BlockSpec((RB, E), lambda i: (i, 0))
  full = lambda a: pl.BlockSpec(a.shape, lambda i: (0,) * a.ndim)
  out = pl.pallas_call(
      _tc_body,
      grid=(NBLK,),
      in_specs=[row, row, row, row,
                full(W1), full(b1), full(W2), full(b2), full(Wf), full(bf)],
      out_specs=pl.BlockSpec((RB, 1), lambda i: (i, 0)),
      out_shape=jax.ShapeDtypeStruct((B, 1), jnp.float32),
  )(gu, gi, mu, mi, W1, b1, W2, b2, Wf, bf)
  return out.reshape(B)


def kernel(user_ids, item_ids, gmf_user_table, gmf_item_table,
           mlp_user_table, mlp_item_table, W1, b1, W2, b2, Wf, bf):
  uid = user_ids.astype(jnp.int32)
  iid = item_ids.astype(jnp.int32)
  t3 = lambda t: t.reshape(t.shape[0] // 8, 8, E)
  gu, gi, mu, mi = _sc_gather4(uid, iid,
                               t3(gmf_user_table), t3(gmf_item_table),
                               t3(mlp_user_table), t3(mlp_item_table))
  return _tc_mlp(gu, gi, mu, mi,
                 W1, b1.reshape(1, 64), W2, b2.reshape(1, 32),
                 Wf, bf.reshape(1, 1))
